# Initial kernel scaffold; baseline (speedup 1.0000x reference)
import jax
import jax.numpy as jnp
from jax.experimental import pallas as pl


def kernel(x, relative_embeddings):
    raise NotImplementedError("write your pallas kernel here")


# speedup vs baseline: 2.5768x; 2.5768x over previous
import functools

import jax
import jax.numpy as jnp
from jax import lax
from jax.experimental import pallas as pl
from jax.experimental.pallas import tpu as pltpu
from jax.experimental.pallas import tpu_sc as plsc


def _sc_relpos(seq: int, d: int, num_rel: int):
    max_len = (num_rel + 1) // 2
    win_start = max_len - seq
    win_rows = 2 * seq
    info = plsc.get_sparse_core_info()
    nc, ns = info.num_cores, info.num_subcores
    nw = nc * ns
    assert seq % nw == 0
    rows_per_w = seq // nw

    mesh = plsc.VectorSubcoreMesh(core_axis_name="c", subcore_axis_name="s")

    @functools.partial(
        pl.kernel,
        mesh=mesh,
        out_type=jax.ShapeDtypeStruct((seq * seq, d), jnp.float32),
        scratch_types=[
            pltpu.VMEM_SHARED((win_rows, d), jnp.float32),
            pltpu.SemaphoreType.DMA,
        ],
        compiler_params=pltpu.CompilerParams(use_tc_tiling_on_sc=False),
    )
    def body(table_hbm, out_hbm, window, sem):
        cid = lax.axis_index("c")
        sid = lax.axis_index("s")

        @pl.when(sid == 0)
        def _load():
            pltpu.sync_copy(table_hbm.at[pl.ds(win_start, win_rows)], window)

        plsc.subcore_barrier()

        wid = sid * nc + cid
        base = wid * rows_per_w
        copies = []
        for k in range(rows_per_w):
            i = base + k
            c = pltpu.make_async_copy(
                window.at[pl.ds(seq - 1 - i, seq)],
                out_hbm.at[pl.ds(i * seq, seq)],
                sem,
            )
            c.start()
            copies.append(c)
        for c in copies:
            c.wait()

    return body


def kernel(x, relative_embeddings):
    seq = x.shape[0]
    d = relative_embeddings.shape[1]
    num_rel = relative_embeddings.shape[0]
    out = _sc_relpos(seq, d, num_rel)(relative_embeddings)
    return out.reshape(seq, seq, d)
